# Initial kernel scaffold; baseline (speedup 1.0000x reference)
import jax
import jax.numpy as jnp
from jax.experimental import pallas as pl


def kernel(logits, temperatures, top_ps, top_ks, min_ps, greedy_indices, random_indices, q):
    raise NotImplementedError("write your pallas kernel here")


# speedup vs baseline: 43.4782x; 43.4782x over previous
import jax
import jax.numpy as jnp
import numpy as np
from jax.experimental import pallas as pl

_R = 8
_I32_MIN = np.int32(-2147483648)
_I32_MAX = np.int32(2147483647)
_INCS = [np.int32(np.uint32(1 << b).astype(np.int32)) for b in range(31, -1, -1)]


def _sampler_kernel(temp_ref, topp_ref, topk_ref, minp_ref, logits_ref, q_ref,
                    lp_ref, gidx_ref, ridx_ref, *, n_greedy_blocks):
    i = pl.program_id(0)
    x = logits_ref[...] / temp_ref[...]
    m = jnp.max(x, axis=1, keepdims=True)
    w = jnp.exp(x - m)
    bits = jax.lax.bitcast_convert_type(x, jnp.int32)
    c = jnp.where(bits < 0, bits ^ np.int32(0x7FFFFFFF), bits)

    k = topk_ref[...]
    t = jnp.full(k.shape, _I32_MIN, jnp.int32)
    for inc in _INCS:
        cand = t + inc
        cnt = jnp.sum((c >= cand).astype(jnp.int32), axis=1, keepdims=True)
        t = jnp.where((cand > t) & (cnt >= k), cand, t)
    keep_k = c >= t
    w_k = jnp.where(keep_k, w, 0.0)
    zk = jnp.sum(w_k, axis=1, keepdims=True)
    pz = topp_ref[...] * zk

    t2 = t - 1
    for inc in _INCS:
        cand = t2 + inc
        s = jnp.sum(jnp.where(c > cand, w_k, 0.0), axis=1, keepdims=True)
        t2 = jnp.where((cand > t2) & (s >= pz), cand, t2)

    kept = keep_k & (c > t2) & (w >= minp_ref[...])
    w_f = jnp.where(kept, w, 0.0)
    zf = jnp.sum(w_f, axis=1, keepdims=True)
    lp_ref[...] = jnp.where(kept, x - m - jnp.log(zf), -jnp.inf)

    iota = jax.lax.broadcasted_iota(jnp.int32, x.shape, 1)

    @pl.when(i < n_greedy_blocks)
    def _greedy():
        gidx_ref[...] = jnp.min(jnp.where(x == m, iota, _I32_MAX),
                                axis=1, keepdims=True)

    @pl.when(i >= n_greedy_blocks)
    def _random():
        rat = w_f / q_ref[...]
        nan_mask = jnp.isnan(rat)
        nan_idx = jnp.min(jnp.where(nan_mask, iota, _I32_MAX),
                          axis=1, keepdims=True)
        rat_c = jnp.where(nan_mask, -jnp.inf, rat)
        rmax = jnp.max(rat_c, axis=1, keepdims=True)
        ridx = jnp.min(jnp.where(rat_c == rmax, iota, _I32_MAX),
                       axis=1, keepdims=True)
        ridx_ref[...] = jnp.where(nan_idx < _I32_MAX, nan_idx, ridx)


def kernel(logits, temperatures, top_ps, top_ks, min_ps,
           greedy_indices, random_indices, q):
    B, V = logits.shape
    NQ = q.shape[0]
    n_greedy_blocks = (B - NQ) // _R
    import functools
    body = functools.partial(_sampler_kernel, n_greedy_blocks=n_greedy_blocks)
    row_spec = pl.BlockSpec((_R, 1), lambda i: (i, 0))
    big_spec = pl.BlockSpec((_R, V), lambda i: (i, 0))
    q_spec = pl.BlockSpec((_R, V),
                          lambda i: (jnp.maximum(i - n_greedy_blocks, 0), 0))
    lp, gidx, ridx = pl.pallas_call(
        body,
        grid=(B // _R,),
        in_specs=[row_spec, row_spec, row_spec, row_spec, big_spec, q_spec],
        out_specs=[big_spec,
                   pl.BlockSpec((_R, 1), lambda i: (i, 0)),
                   pl.BlockSpec((_R, 1), lambda i: (i, 0))],
        out_shape=[jax.ShapeDtypeStruct((B, V), jnp.float32),
                   jax.ShapeDtypeStruct((B, 1), jnp.int32),
                   jax.ShapeDtypeStruct((B, 1), jnp.int32)],
    )(temperatures.astype(jnp.float32).reshape(B, 1),
      top_ps.astype(jnp.float32).reshape(B, 1),
      top_ks.astype(jnp.int32).reshape(B, 1),
      min_ps.astype(jnp.float32).reshape(B, 1),
      logits.astype(jnp.float32), q)
    greedy = jnp.take(gidx[:, 0], greedy_indices)
    multinomial = jnp.take(ridx[:, 0], random_indices).reshape(-1, 1)
    return (lp, greedy, multinomial)
